# baseline (speedup 1.0000x reference)
import functools

import jax
import jax.numpy as jnp
from jax import lax
from jax.experimental import pallas as pl
from jax.experimental.pallas import tpu as pltpu
from jax.experimental.pallas import tpu_sc as plsc

N_NODES = 10000
N_EDGES = 320000
C = 128

NC = 2
NS = 16
NW = NC * NS

W = 64
NCHUNK = N_EDGES // W
CPW = NCHUNK // NW
CPW_EXTRA = NCHUNK - CPW * NW
N_PAD = 10240
NODE_CHUNK = N_PAD // NS


def _sc_aggregate(x, e3):
  @functools.partial(
      pl.kernel,
      mesh=plsc.VectorSubcoreMesh(core_axis_name="c", subcore_axis_name="s"),
      compiler_params=pltpu.CompilerParams(use_tc_tiling_on_sc=False),
      out_type=[
          jax.ShapeDtypeStruct((NC, N_PAD, C), jnp.float32),
          jax.ShapeDtypeStruct((NC, N_PAD, 16), jnp.float32),
      ],
      scratch_types=[
          pltpu.VMEM(((CPW + 1) * W,), jnp.int32),
          pltpu.VMEM(((CPW + 1) * W,), jnp.int32),
          pltpu.VMEM((2, W, C), jnp.float32),
          pltpu.VMEM((W, 16), jnp.float32),
          pltpu.VMEM_SHARED((N_PAD, C), jnp.float32),
          pltpu.VMEM_SHARED((N_PAD, 16), jnp.float32),
          pltpu.SemaphoreType.DMA((2,)),
          pltpu.SemaphoreType.DMA,
      ],
  )
  def k(x_hbm, e3_hbm, acc_out, cnt_out, sidx, didx, rows2, ones_v,
        acc_sh, cnt_sh, sem2, csem):
    c = lax.axis_index("c")
    s = lax.axis_index("s")
    w = s * NC + c

    def fill_zero(i, _):
      def fill_lane(j, _):
        rows2[0, i, pl.ds(j * 16, 16)] = jnp.zeros((16,), jnp.float32)
        return 0
      lax.fori_loop(0, C // 16, fill_lane, 0)
      ones_v[i] = jnp.zeros((16,), jnp.float32)
      return 0
    lax.fori_loop(0, W, fill_zero, 0)

    nb = pl.multiple_of(s * NODE_CHUNK, 8)

    def zero_chunk(j, _):
      off = pl.multiple_of(nb + j * W, 8)
      pltpu.sync_copy(rows2.at[0], acc_sh.at[pl.ds(off, W)])
      pltpu.sync_copy(ones_v, cnt_sh.at[pl.ds(off, W)])
      return 0
    lax.fori_loop(0, NODE_CHUNK // W, zero_chunk, 0)

    def fill_one(i, _):
      ones_v[i] = jnp.ones((16,), jnp.float32)
      return 0
    lax.fori_loop(0, W, fill_one, 0)

    nchunks = CPW + jnp.where(w < CPW_EXTRA, 1, 0)
    base = (CPW * w + jnp.minimum(w, CPW_EXTRA)) * W
    pltpu.sync_copy(e3_hbm.at[0, pl.ds(base, CPW * W)],
                    sidx.at[pl.ds(0, CPW * W)])
    pltpu.sync_copy(e3_hbm.at[1, pl.ds(base, CPW * W)],
                    didx.at[pl.ds(0, CPW * W)])

    @pl.when(w < CPW_EXTRA)
    def _():
      pltpu.sync_copy(e3_hbm.at[0, pl.ds(base + CPW * W, W)],
                      sidx.at[pl.ds(CPW * W, W)])
      pltpu.sync_copy(e3_hbm.at[1, pl.ds(base + CPW * W, W)],
                      didx.at[pl.ds(CPW * W, W)])

    plsc.subcore_barrier()

    def step(i, carry):
      b = jnp.bitwise_and(i, 1)

      @pl.when(i < nchunks)
      def _():
        pltpu.async_copy(x_hbm.at[sidx.at[pl.ds(i * W, W)]],
                         rows2.at[b], sem2.at[b])

      @pl.when(i > 0)
      def _():
        pb = jnp.bitwise_and(i - 1, 1)
        dix = didx.at[pl.ds((i - 1) * W, W)]
        pltpu.make_async_copy(
            x_hbm.at[pl.ds(0, W)], rows2.at[pb], sem2.at[pb]).wait()
        pltpu.sync_copy(rows2.at[pb], acc_sh.at[dix], add=True)
        pltpu.async_copy(ones_v, cnt_sh.at[dix], csem, add=True)
      return carry

    lax.fori_loop(0, nchunks + 1, step, 0)

    def drain(i, carry):
      pltpu.make_async_copy(
          x_hbm.at[pl.ds(0, W), pl.ds(0, 16)], ones_v, csem).wait()
      return carry

    lax.fori_loop(0, nchunks, drain, 0)

    plsc.subcore_barrier()

    pltpu.sync_copy(acc_sh.at[pl.ds(nb, NODE_CHUNK)],
                    acc_out.at[c, pl.ds(nb, NODE_CHUNK)])
    pltpu.sync_copy(cnt_sh.at[pl.ds(nb, NODE_CHUNK)],
                    cnt_out.at[c, pl.ds(nb, NODE_CHUNK)])

  return k(x, e3)


def _tc_self_body(x, wr, b, y):
  y[...] = (jnp.dot(x[...], wr[...], preferred_element_type=jnp.float32)
            + b[...])


def _tc_self(x, wrT, b):
  R = 2000
  return pl.pallas_call(
      _tc_self_body,
      grid=(N_NODES // R,),
      in_specs=[
          pl.BlockSpec((R, C), lambda i: (i, 0)),
          pl.BlockSpec((C, C), lambda i: (0, 0)),
          pl.BlockSpec((1, C), lambda i: (0, 0)),
      ],
      out_specs=pl.BlockSpec((R, C), lambda i: (i, 0)),
      out_shape=jax.ShapeDtypeStruct((N_NODES, C), jnp.float32),
  )(x, wrT, b)


def _tc_body(pacc, pcnt, y, wl, out):
  acc = pacc[0] + pacc[1]
  cnt = pcnt[0] + pcnt[1]
  mean = acc / jnp.maximum(cnt[:, 0:1], 1.0)
  out[...] = jnp.maximum(
      jnp.dot(mean, wl[...], preferred_element_type=jnp.float32) + y[...], 0.0)


def _tc_finish(pacc, pcnt, y, wlT):
  R = 2000
  grid = (N_NODES // R,)
  return pl.pallas_call(
      _tc_body,
      grid=grid,
      in_specs=[
          pl.BlockSpec((NC, R, C), lambda i: (0, i, 0)),
          pl.BlockSpec((NC, R, 16), lambda i: (0, i, 0)),
          pl.BlockSpec((R, C), lambda i: (i, 0)),
          pl.BlockSpec((C, C), lambda i: (0, 0)),
      ],
      out_specs=pl.BlockSpec((R, C), lambda i: (i, 0)),
      out_shape=jax.ShapeDtypeStruct((N_NODES, C), jnp.float32),
  )(pacc, pcnt, y, wlT)


def kernel(x, edge_index, W_l, b_l, W_r):
  pacc, pcnt = _sc_aggregate(x, edge_index)
  y = _tc_self(x, W_r.T, b_l.reshape(1, C))
  return _tc_finish(pacc, pcnt, y, W_l.T)


# speedup vs baseline: 1.0005x; 1.0005x over previous
import functools

import jax
import jax.numpy as jnp
from jax import lax
from jax.experimental import pallas as pl
from jax.experimental.pallas import tpu as pltpu
from jax.experimental.pallas import tpu_sc as plsc

N_NODES = 10000
N_EDGES = 320000
C = 128

NC = 2
NS = 16
NW = NC * NS

W = 64
NCHUNK = N_EDGES // W
CPW = NCHUNK // NW
CPW_EXTRA = NCHUNK - CPW * NW
N_PAD = 10240
NODE_CHUNK = N_PAD // NS


def _sc_aggregate(x, e3):
  @functools.partial(
      pl.kernel,
      mesh=plsc.VectorSubcoreMesh(core_axis_name="c", subcore_axis_name="s"),
      compiler_params=pltpu.CompilerParams(use_tc_tiling_on_sc=False),
      out_type=[
          jax.ShapeDtypeStruct((NC, N_PAD, C), jnp.float32),
          jax.ShapeDtypeStruct((NC, N_PAD, 16), jnp.float32),
      ],
      scratch_types=[
          pltpu.VMEM(((CPW + 1) * W,), jnp.int32),
          pltpu.VMEM(((CPW + 1) * W,), jnp.int32),
          pltpu.VMEM((2, W, C), jnp.float32),
          pltpu.VMEM((W, 16), jnp.float32),
          pltpu.VMEM_SHARED((N_PAD, C), jnp.float32),
          pltpu.VMEM_SHARED((N_PAD, 16), jnp.float32),
          pltpu.SemaphoreType.DMA((2,)),
          pltpu.SemaphoreType.DMA,
      ],
  )
  def k(x_hbm, e3_hbm, acc_out, cnt_out, sidx, didx, rows2, ones_v,
        acc_sh, cnt_sh, sem2, csem):
    c = lax.axis_index("c")
    s = lax.axis_index("s")
    w = s * NC + c

    def fill_zero(i, _):
      def fill_lane(j, _):
        rows2[0, i, pl.ds(j * 16, 16)] = jnp.zeros((16,), jnp.float32)
        return 0
      lax.fori_loop(0, C // 16, fill_lane, 0)
      ones_v[i] = jnp.zeros((16,), jnp.float32)
      return 0
    lax.fori_loop(0, W, fill_zero, 0)

    nb = pl.multiple_of(s * NODE_CHUNK, 8)

    def zero_chunk(j, _):
      off = pl.multiple_of(nb + j * W, 8)
      pltpu.sync_copy(rows2.at[0], acc_sh.at[pl.ds(off, W)])
      pltpu.sync_copy(ones_v, cnt_sh.at[pl.ds(off, W)])
      return 0
    lax.fori_loop(0, NODE_CHUNK // W, zero_chunk, 0)

    def fill_one(i, _):
      ones_v[i] = jnp.ones((16,), jnp.float32)
      return 0
    lax.fori_loop(0, W, fill_one, 0)

    nchunks = CPW + jnp.where(w < CPW_EXTRA, 1, 0)
    base = (CPW * w + jnp.minimum(w, CPW_EXTRA)) * W
    pltpu.sync_copy(e3_hbm.at[pl.ds(base, CPW * W)],
                    sidx.at[pl.ds(0, CPW * W)])
    pltpu.sync_copy(e3_hbm.at[pl.ds(N_EDGES + base, CPW * W)],
                    didx.at[pl.ds(0, CPW * W)])

    @pl.when(w < CPW_EXTRA)
    def _():
      pltpu.sync_copy(e3_hbm.at[pl.ds(base + CPW * W, W)],
                      sidx.at[pl.ds(CPW * W, W)])
      pltpu.sync_copy(e3_hbm.at[pl.ds(N_EDGES + base + CPW * W, W)],
                      didx.at[pl.ds(CPW * W, W)])

    plsc.subcore_barrier()

    def step(i, carry):
      b = jnp.bitwise_and(i, 1)

      @pl.when(i < nchunks)
      def _():
        pltpu.async_copy(x_hbm.at[sidx.at[pl.ds(i * W, W)]],
                         rows2.at[b], sem2.at[b])

      @pl.when(i > 0)
      def _():
        pb = jnp.bitwise_and(i - 1, 1)
        dix = didx.at[pl.ds((i - 1) * W, W)]
        pltpu.make_async_copy(
            x_hbm.at[pl.ds(0, W)], rows2.at[pb], sem2.at[pb]).wait()
        pltpu.sync_copy(rows2.at[pb], acc_sh.at[dix], add=True)
        pltpu.async_copy(ones_v, cnt_sh.at[dix], csem, add=True)
      return carry

    lax.fori_loop(0, nchunks + 1, step, 0)

    def drain(i, carry):
      pltpu.make_async_copy(
          x_hbm.at[pl.ds(0, W), pl.ds(0, 16)], ones_v, csem).wait()
      return carry

    lax.fori_loop(0, nchunks, drain, 0)

    plsc.subcore_barrier()

    pltpu.sync_copy(acc_sh.at[pl.ds(nb, NODE_CHUNK)],
                    acc_out.at[c, pl.ds(nb, NODE_CHUNK)])
    pltpu.sync_copy(cnt_sh.at[pl.ds(nb, NODE_CHUNK)],
                    cnt_out.at[c, pl.ds(nb, NODE_CHUNK)])

  return k(x, e3)


def _tc_self_body(x, wr, b, y):
  y[...] = (jnp.dot(x[...], wr[...], preferred_element_type=jnp.float32)
            + b[...])


def _tc_self(x, wrT, b):
  R = 2000
  return pl.pallas_call(
      _tc_self_body,
      grid=(N_NODES // R,),
      in_specs=[
          pl.BlockSpec((R, C), lambda i: (i, 0)),
          pl.BlockSpec((C, C), lambda i: (0, 0)),
          pl.BlockSpec((1, C), lambda i: (0, 0)),
      ],
      out_specs=pl.BlockSpec((R, C), lambda i: (i, 0)),
      out_shape=jax.ShapeDtypeStruct((N_NODES, C), jnp.float32),
  )(x, wrT, b)


def _tc_body(pacc, pcnt, y, wl, out):
  acc = pacc[0] + pacc[1]
  cnt = pcnt[0] + pcnt[1]
  mean = acc / jnp.maximum(cnt[:, 0:1], 1.0)
  out[...] = jnp.maximum(
      jnp.dot(mean, wl[...], preferred_element_type=jnp.float32) + y[...], 0.0)


def _tc_finish(pacc, pcnt, y, wlT):
  R = 2000
  grid = (N_NODES // R,)
  return pl.pallas_call(
      _tc_body,
      grid=grid,
      in_specs=[
          pl.BlockSpec((NC, R, C), lambda i: (0, i, 0)),
          pl.BlockSpec((NC, R, 16), lambda i: (0, i, 0)),
          pl.BlockSpec((R, C), lambda i: (i, 0)),
          pl.BlockSpec((C, C), lambda i: (0, 0)),
      ],
      out_specs=pl.BlockSpec((R, C), lambda i: (i, 0)),
      out_shape=jax.ShapeDtypeStruct((N_NODES, C), jnp.float32),
  )(pacc, pcnt, y, wlT)


def kernel(x, edge_index, W_l, b_l, W_r):
  pacc, pcnt = _sc_aggregate(x, edge_index.reshape(2 * N_EDGES))
  y = _tc_self(x, W_r.T, b_l.reshape(1, C))
  return _tc_finish(pacc, pcnt, y, W_l.T)


# speedup vs baseline: 1.0457x; 1.0452x over previous
import functools

import jax
import jax.numpy as jnp
from jax import lax
from jax.experimental import pallas as pl
from jax.experimental.pallas import tpu as pltpu
from jax.experimental.pallas import tpu_sc as plsc

N_NODES = 10000
N_EDGES = 320000
C = 128

NC = 2
NS = 16
NW = NC * NS

W = 64
NCHUNK = N_EDGES // W
CPW = NCHUNK // NW
CPW_EXTRA = NCHUNK - CPW * NW
N_PAD = 10240
NODE_CHUNK = N_PAD // NS


def _sc_aggregate(x, e3):
  @functools.partial(
      pl.kernel,
      mesh=plsc.VectorSubcoreMesh(core_axis_name="c", subcore_axis_name="s"),
      compiler_params=pltpu.CompilerParams(use_tc_tiling_on_sc=False),
      out_type=[
          jax.ShapeDtypeStruct((NC, N_PAD, C), jnp.float32),
          jax.ShapeDtypeStruct((NC, N_PAD, 16), jnp.float32),
      ],
      scratch_types=[
          pltpu.VMEM(((CPW + 1) * W,), jnp.int32),
          pltpu.VMEM(((CPW + 1) * W,), jnp.int32),
          pltpu.VMEM((2, W, C), jnp.float32),
          pltpu.VMEM((W, 16), jnp.float32),
          pltpu.VMEM_SHARED((N_PAD, C), jnp.float32),
          pltpu.VMEM_SHARED((N_PAD, 16), jnp.float32),
          pltpu.SemaphoreType.DMA((2,)),
          pltpu.SemaphoreType.DMA,
      ],
  )
  def k(x_hbm, e3_hbm, acc_out, cnt_out, sidx, didx, rows2, ones_v,
        acc_sh, cnt_sh, sem2, csem):
    c = lax.axis_index("c")
    s = lax.axis_index("s")
    w = s * NC + c

    def fill_zero(i, _):
      def fill_lane(j, _):
        rows2[0, i, pl.ds(j * 16, 16)] = jnp.zeros((16,), jnp.float32)
        return 0
      lax.fori_loop(0, C // 16, fill_lane, 0)
      ones_v[i] = jnp.zeros((16,), jnp.float32)
      return 0
    lax.fori_loop(0, W, fill_zero, 0)

    nb = pl.multiple_of(s * NODE_CHUNK, 8)

    def zero_chunk(j, _):
      off = pl.multiple_of(nb + j * W, 8)
      pltpu.sync_copy(rows2.at[0], acc_sh.at[pl.ds(off, W)])
      pltpu.sync_copy(ones_v, cnt_sh.at[pl.ds(off, W)])
      return 0
    lax.fori_loop(0, NODE_CHUNK // W, zero_chunk, 0)

    def fill_one(i, _):
      ones_v[i] = jnp.ones((16,), jnp.float32)
      return 0
    lax.fori_loop(0, W, fill_one, 0)

    nchunks = CPW + jnp.where(w < CPW_EXTRA, 1, 0)
    base = (CPW * w + jnp.minimum(w, CPW_EXTRA)) * W
    pltpu.sync_copy(e3_hbm.at[pl.ds(base, CPW * W)],
                    sidx.at[pl.ds(0, CPW * W)])
    pltpu.sync_copy(e3_hbm.at[pl.ds(N_EDGES + base, CPW * W)],
                    didx.at[pl.ds(0, CPW * W)])

    @pl.when(w < CPW_EXTRA)
    def _():
      pltpu.sync_copy(e3_hbm.at[pl.ds(base + CPW * W, W)],
                      sidx.at[pl.ds(CPW * W, W)])
      pltpu.sync_copy(e3_hbm.at[pl.ds(N_EDGES + base + CPW * W, W)],
                      didx.at[pl.ds(CPW * W, W)])

    plsc.subcore_barrier()

    def step(i, carry):
      b = jnp.bitwise_and(i, 1)

      @pl.when(i < nchunks)
      def _():
        pltpu.async_copy(x_hbm.at[sidx.at[pl.ds(i * W, W)]],
                         rows2.at[b], sem2.at[b])

      @pl.when(i > 0)
      def _():
        pb = jnp.bitwise_and(i - 1, 1)
        dix = didx.at[pl.ds((i - 1) * W, W)]
        pltpu.make_async_copy(
            x_hbm.at[pl.ds(0, W)], rows2.at[pb], sem2.at[pb]).wait()
        pltpu.sync_copy(rows2.at[pb], acc_sh.at[dix], add=True)
        pltpu.async_copy(ones_v, cnt_sh.at[dix], csem, add=True)
      return carry

    lax.fori_loop(0, nchunks + 1, step, 0)

    def drain(i, carry):
      pltpu.make_async_copy(
          x_hbm.at[pl.ds(0, W), pl.ds(0, 16)], ones_v, csem).wait()
      return carry

    lax.fori_loop(0, nchunks, drain, 0)

    plsc.subcore_barrier()

    pltpu.sync_copy(acc_sh.at[pl.ds(nb, NODE_CHUNK)],
                    acc_out.at[c, pl.ds(nb, NODE_CHUNK)])
    pltpu.sync_copy(cnt_sh.at[pl.ds(nb, NODE_CHUNK)],
                    cnt_out.at[c, pl.ds(nb, NODE_CHUNK)])

  return k(x, e3)


def _tc_self_body(x, wr, b, y):
  y[...] = (jnp.dot(x[...], wr[...], preferred_element_type=jnp.float32)
            + b[...])


def _tc_self(x, wrT, b):
  R = 2000
  return pl.pallas_call(
      _tc_self_body,
      grid=(N_NODES // R,),
      in_specs=[
          pl.BlockSpec((R, C), lambda i: (i, 0)),
          pl.BlockSpec((C, C), lambda i: (0, 0)),
          pl.BlockSpec((1, C), lambda i: (0, 0)),
      ],
      out_specs=pl.BlockSpec((R, C), lambda i: (i, 0)),
      out_shape=jax.ShapeDtypeStruct((N_NODES, C), jnp.float32),
  )(x, wrT, b)


def _tc_body(pacc, pcnt, y, wl, out):
  acc = pacc[0] + pacc[1]
  cnt = pcnt[0] + pcnt[1]
  R = acc.shape[0]
  f32 = jnp.float32
  ri = lax.broadcasted_iota(jnp.int32, (R, R // 8), 0) // 8
  ci = lax.broadcasted_iota(jnp.int32, (R, R // 8), 1)
  P = jnp.where(ri == ci, 1.0, 0.0).astype(f32)
  rep = jnp.dot(P, cnt, preferred_element_type=f32)
  li = lax.broadcasted_iota(jnp.int32, (R, C), 1) // 16
  ni = lax.broadcasted_iota(jnp.int32, (R, C), 0) % 8
  masked = jnp.where(li == ni, rep, 0.0)
  spread = jnp.full((C, C), 1.0 / 16.0, dtype=f32)
  d = jnp.dot(masked, spread, preferred_element_type=f32)
  mean = acc / jnp.maximum(d, 1.0)
  out[...] = jnp.maximum(
      jnp.dot(mean, wl[...], preferred_element_type=jnp.float32) + y[...], 0.0)


def _tc_finish(pacc, pcnt, y, wlT):
  R = 2048
  grid = (N_PAD // R,)
  return pl.pallas_call(
      _tc_body,
      grid=grid,
      in_specs=[
          pl.BlockSpec((NC, R, C), lambda i: (0, i, 0)),
          pl.BlockSpec((NC, R * 16 // C, C), lambda i: (0, i, 0)),
          pl.BlockSpec((R, C), lambda i: (i, 0)),
          pl.BlockSpec((C, C), lambda i: (0, 0)),
      ],
      out_specs=pl.BlockSpec((R, C), lambda i: (i, 0)),
      out_shape=jax.ShapeDtypeStruct((N_NODES, C), jnp.float32),
  )(pacc, pcnt, y, wlT)


def kernel(x, edge_index, W_l, b_l, W_r):
  pacc, pcnt = _sc_aggregate(x, edge_index.reshape(2 * N_EDGES))
  pcnt128 = pcnt.reshape(NC, N_PAD * 16 // C, C)
  y = _tc_self(x, W_r.T, b_l.reshape(1, C))
  return _tc_finish(pacc, pcnt128, y, W_l.T)


# speedup vs baseline: 1.0712x; 1.0244x over previous
import functools

import jax
import jax.numpy as jnp
from jax import lax
from jax.experimental import pallas as pl
from jax.experimental.pallas import tpu as pltpu
from jax.experimental.pallas import tpu_sc as plsc

N_NODES = 10000
N_EDGES = 320000
C = 128

NC = 2
NS = 16
NW = NC * NS

W = 64
NCHUNK = N_EDGES // W
CPW = NCHUNK // NW
CPW_EXTRA = NCHUNK - CPW * NW
N_PAD = 10240
NODE_CHUNK = N_PAD // NS


def _sc_aggregate(x, e3):
  @functools.partial(
      pl.kernel,
      mesh=plsc.VectorSubcoreMesh(core_axis_name="c", subcore_axis_name="s"),
      compiler_params=pltpu.CompilerParams(use_tc_tiling_on_sc=False),
      out_type=[
          jax.ShapeDtypeStruct((NC, N_PAD, C), jnp.float32),
          jax.ShapeDtypeStruct((NC, N_PAD, 16), jnp.float32),
      ],
      scratch_types=[
          pltpu.VMEM(((CPW + 1) * W,), jnp.int32),
          pltpu.VMEM(((CPW + 1) * W,), jnp.int32),
          pltpu.VMEM((2, W, C), jnp.float32),
          pltpu.VMEM((W, 16), jnp.float32),
          pltpu.VMEM_SHARED((N_PAD, C), jnp.float32),
          pltpu.VMEM_SHARED((N_PAD, 16), jnp.float32),
          pltpu.SemaphoreType.DMA((2,)),
          pltpu.SemaphoreType.DMA,
      ],
  )
  def k(x_hbm, e3_hbm, acc_out, cnt_out, sidx, didx, rows2, ones_v,
        acc_sh, cnt_sh, sem2, csem):
    c = lax.axis_index("c")
    s = lax.axis_index("s")
    w = s * NC + c

    def fill_zero(i, _):
      def fill_lane(j, _):
        rows2[0, i, pl.ds(j * 16, 16)] = jnp.zeros((16,), jnp.float32)
        return 0
      lax.fori_loop(0, C // 16, fill_lane, 0)
      ones_v[i] = jnp.zeros((16,), jnp.float32)
      return 0
    lax.fori_loop(0, W, fill_zero, 0)

    nb = pl.multiple_of(s * NODE_CHUNK, 8)

    def zero_chunk(j, _):
      off = pl.multiple_of(nb + j * W, 8)
      pltpu.async_copy(rows2.at[0], acc_sh.at[pl.ds(off, W)], csem)
      pltpu.async_copy(ones_v, cnt_sh.at[pl.ds(off, W)], csem)
      return 0
    lax.fori_loop(0, NODE_CHUNK // W, zero_chunk, 0)

    nchunks = CPW + jnp.where(w < CPW_EXTRA, 1, 0)
    base = (CPW * w + jnp.minimum(w, CPW_EXTRA)) * W
    pltpu.sync_copy(e3_hbm.at[pl.ds(base, CPW * W)],
                    sidx.at[pl.ds(0, CPW * W)])
    pltpu.sync_copy(e3_hbm.at[pl.ds(N_EDGES + base, CPW * W)],
                    didx.at[pl.ds(0, CPW * W)])

    @pl.when(w < CPW_EXTRA)
    def _():
      pltpu.sync_copy(e3_hbm.at[pl.ds(base + CPW * W, W)],
                      sidx.at[pl.ds(CPW * W, W)])
      pltpu.sync_copy(e3_hbm.at[pl.ds(N_EDGES + base + CPW * W, W)],
                      didx.at[pl.ds(CPW * W, W)])

    def zdrain(j, _):
      pltpu.make_async_copy(x_hbm.at[pl.ds(0, W)], rows2.at[0], csem).wait()
      pltpu.make_async_copy(
          x_hbm.at[pl.ds(0, W), pl.ds(0, 16)], ones_v, csem).wait()
      return 0
    lax.fori_loop(0, NODE_CHUNK // W, zdrain, 0)

    def fill_one(i, _):
      ones_v[i] = jnp.ones((16,), jnp.float32)
      return 0
    lax.fori_loop(0, W, fill_one, 0)

    plsc.subcore_barrier()

    def step(i, carry):
      b = jnp.bitwise_and(i, 1)

      @pl.when(i < nchunks)
      def _():
        pltpu.async_copy(x_hbm.at[sidx.at[pl.ds(i * W, W)]],
                         rows2.at[b], sem2.at[b])

      @pl.when(i > 0)
      def _():
        pb = jnp.bitwise_and(i - 1, 1)
        dix = didx.at[pl.ds((i - 1) * W, W)]
        pltpu.make_async_copy(
            x_hbm.at[pl.ds(0, W)], rows2.at[pb], sem2.at[pb]).wait()
        pltpu.sync_copy(rows2.at[pb], acc_sh.at[dix], add=True)
        pltpu.async_copy(ones_v, cnt_sh.at[dix], csem, add=True)
      return carry

    lax.fori_loop(0, nchunks + 1, step, 0)

    def drain(i, carry):
      pltpu.make_async_copy(
          x_hbm.at[pl.ds(0, W), pl.ds(0, 16)], ones_v, csem).wait()
      return carry

    lax.fori_loop(0, nchunks, drain, 0)

    plsc.subcore_barrier()

    pltpu.sync_copy(acc_sh.at[pl.ds(nb, NODE_CHUNK)],
                    acc_out.at[c, pl.ds(nb, NODE_CHUNK)])
    pltpu.sync_copy(cnt_sh.at[pl.ds(nb, NODE_CHUNK)],
                    cnt_out.at[c, pl.ds(nb, NODE_CHUNK)])

  return k(x, e3)


def _tc_self_body(x, wr, b, y):
  y[...] = (jnp.dot(x[...], wr[...], preferred_element_type=jnp.float32)
            + b[...])


def _tc_self(x, wrT, b):
  R = 2000
  return pl.pallas_call(
      _tc_self_body,
      grid=(N_NODES // R,),
      in_specs=[
          pl.BlockSpec((R, C), lambda i: (i, 0)),
          pl.BlockSpec((C, C), lambda i: (0, 0)),
          pl.BlockSpec((1, C), lambda i: (0, 0)),
      ],
      out_specs=pl.BlockSpec((R, C), lambda i: (i, 0)),
      out_shape=jax.ShapeDtypeStruct((N_NODES, C), jnp.float32),
  )(x, wrT, b)


def _tc_body(pacc, pcnt, y, wl, out):
  acc = pacc[0] + pacc[1]
  cnt = pcnt[0] + pcnt[1]
  R = acc.shape[0]
  f32 = jnp.float32
  ri = lax.broadcasted_iota(jnp.int32, (R, R // 8), 0) // 8
  ci = lax.broadcasted_iota(jnp.int32, (R, R // 8), 1)
  P = jnp.where(ri == ci, 1.0, 0.0).astype(f32)
  rep = jnp.dot(P, cnt, preferred_element_type=f32)
  li = lax.broadcasted_iota(jnp.int32, (R, C), 1) // 16
  ni = lax.broadcasted_iota(jnp.int32, (R, C), 0) % 8
  masked = jnp.where(li == ni, rep, 0.0)
  spread = jnp.full((C, C), 1.0 / 16.0, dtype=f32)
  d = jnp.dot(masked, spread, preferred_element_type=f32)
  mean = acc / jnp.maximum(d, 1.0)
  out[...] = jnp.maximum(
      jnp.dot(mean, wl[...], preferred_element_type=jnp.float32) + y[...], 0.0)


def _tc_finish(pacc, pcnt, y, wlT):
  R = 2048
  grid = (N_PAD // R,)
  return pl.pallas_call(
      _tc_body,
      grid=grid,
      in_specs=[
          pl.BlockSpec((NC, R, C), lambda i: (0, i, 0)),
          pl.BlockSpec((NC, R * 16 // C, C), lambda i: (0, i, 0)),
          pl.BlockSpec((R, C), lambda i: (i, 0)),
          pl.BlockSpec((C, C), lambda i: (0, 0)),
      ],
      out_specs=pl.BlockSpec((R, C), lambda i: (i, 0)),
      out_shape=jax.ShapeDtypeStruct((N_NODES, C), jnp.float32),
  )(pacc, pcnt, y, wlT)


def kernel(x, edge_index, W_l, b_l, W_r):
  pacc, pcnt = _sc_aggregate(x, edge_index.reshape(2 * N_EDGES))
  pcnt128 = pcnt.reshape(NC, N_PAD * 16 // C, C)
  y = _tc_self(x, W_r.T, b_l.reshape(1, C))
  return _tc_finish(pacc, pcnt128, y, W_l.T)
